# Initial kernel scaffold; baseline (speedup 1.0000x reference)
import jax
import jax.numpy as jnp
from jax.experimental import pallas as pl


def kernel(x, sx, edge_index, edge_attr, tW1, tb1, tW2, tb2, sA1, sa1, sA2, sa2, sB1, sb1, sB2, sb2, mW1, mb1, mW2, mb2):
    raise NotImplementedError("write your pallas kernel here")


# speedup vs baseline: 1.7043x; 1.7043x over previous
import functools

import jax
import jax.numpy as jnp
from jax import lax
from jax.experimental import pallas as pl
from jax.experimental.pallas import tpu as pltpu
from jax.experimental.pallas import tpu_sc as plsc

NC, NS = 2, 16
NW = NC * NS


def _dense_pre(x, sx, tW1, tb1, tW2, tb2, sA1, sa1):
    n = x.shape[0]
    bn = 2000
    grid = n // bn

    def body(x_r, sx_r, w1a_r, w1b_r, b1_r, w2_r, b2_r, a1a_r, a1b_r, ba_r,
             x0_r, p1_r):
        xb = x_r[...]
        sxb = sx_r[...]
        h = jnp.maximum(
            jnp.dot(xb, w1a_r[...], preferred_element_type=jnp.float32)
            + jnp.dot(sxb, w1b_r[...], preferred_element_type=jnp.float32)
            + b1_r[...], 0.0)
        x0 = jnp.maximum(
            jnp.dot(h, w2_r[...], preferred_element_type=jnp.float32)
            + b2_r[...], 0.0)
        p1 = (jnp.dot(x0, a1a_r[...], preferred_element_type=jnp.float32)
              + jnp.dot(sxb, a1b_r[...], preferred_element_type=jnp.float32)
              + ba_r[...])
        x0_r[...] = x0
        p1_r[...] = p1

    full = lambda s: pl.BlockSpec(s, lambda i: (0, 0))
    return pl.pallas_call(
        body,
        grid=(grid,),
        in_specs=[
            pl.BlockSpec((bn, 28), lambda i: (i, 0)),
            pl.BlockSpec((bn, 8), lambda i: (i, 0)),
            full((28, 32)), full((8, 32)), full((1, 32)),
            full((32, 32)), full((1, 32)),
            full((32, 32)), full((8, 32)), full((1, 32)),
        ],
        out_specs=[
            pl.BlockSpec((bn, 32), lambda i: (i, 0)),
            pl.BlockSpec((bn, 32), lambda i: (i, 0)),
        ],
        out_shape=[
            jax.ShapeDtypeStruct((n, 32), jnp.float32),
            jax.ShapeDtypeStruct((n, 32), jnp.float32),
        ],
    )(x, sx, tW1[:28], tW1[28:], tb1.reshape(1, 32), tW2, tb2.reshape(1, 32),
      sA1[:32], sA1[32:40], sa1.reshape(1, 32))


def _edge_mlp(g, edge_attr, w1e, b1, w2, b2):
    e, d = g.shape
    be = 8000
    grid = e // be

    def body(g_r, ea_r, w1e_r, b1_r, w2_r, b2_r, y_r):
        z = jnp.maximum(
            g_r[...]
            + jnp.dot(ea_r[...], w1e_r[...], preferred_element_type=jnp.float32)
            + b1_r[...], 0.0)
        y = jnp.maximum(
            jnp.dot(z, w2_r[...], preferred_element_type=jnp.float32)
            + b2_r[...], 0.0)
        y_r[0] = y[:, :32]
        y_r[1] = y[:, 32:]

    full = lambda s: pl.BlockSpec(s, lambda i: (0, 0))
    return pl.pallas_call(
        body,
        grid=(grid,),
        in_specs=[
            pl.BlockSpec((be, d), lambda i: (i, 0)),
            pl.BlockSpec((be, 4), lambda i: (i, 0)),
            full((4, d)), full((1, d)), full((d, 64)), full((1, 64)),
        ],
        out_specs=pl.BlockSpec((2, be, 32), lambda i: (0, i, 0)),
        out_shape=jax.ShapeDtypeStruct((2, e, 32), jnp.float32),
    )(g, edge_attr, w1e, b1.reshape(1, d), w2, b2.reshape(1, 64))


def _dense_mid(s1, cnt, x0, sx, sB1, sb1):
    n = x0.shape[0]
    bn = 2000
    grid = n // bn

    def body(sl_r, sh_r, c_r, x0_r, sx_r, wa_r, wb_r, wc_r, wd_r, b_r, p2_r):
        r = 1.0 / jnp.maximum(c_r[...], 1.0)
        o1l = sl_r[...] * r
        o1h = sh_r[...] * r
        p2_r[...] = (
            jnp.dot(o1l, wa_r[...], preferred_element_type=jnp.float32)
            + jnp.dot(o1h, wb_r[...], preferred_element_type=jnp.float32)
            + jnp.dot(x0_r[...], wc_r[...], preferred_element_type=jnp.float32)
            + jnp.dot(sx_r[...], wd_r[...], preferred_element_type=jnp.float32)
            + b_r[...])

    full = lambda s: pl.BlockSpec(s, lambda i: (0, 0))
    return pl.pallas_call(
        body,
        grid=(grid,),
        in_specs=[
            pl.BlockSpec((bn, 32), lambda i: (i, 0)),
            pl.BlockSpec((bn, 32), lambda i: (i, 0)),
            pl.BlockSpec((bn, 1), lambda i: (i, 0)),
            pl.BlockSpec((bn, 32), lambda i: (i, 0)),
            pl.BlockSpec((bn, 8), lambda i: (i, 0)),
            full((32, 64)), full((32, 64)), full((32, 64)), full((8, 64)),
            full((1, 64)),
        ],
        out_specs=pl.BlockSpec((bn, 64), lambda i: (i, 0)),
        out_shape=jax.ShapeDtypeStruct((n, 64), jnp.float32),
    )(s1[0], s1[1], cnt.reshape(n, 1), x0, sx,
      sB1[:32], sB1[32:64], sB1[64:96], sB1[96:104], sb1.reshape(1, 64))


def _dense_head(s2, cnt, x0, sx, mW1, mb1, mW2, mb2):
    n = x0.shape[0]
    bn = 2000
    grid = n // bn

    def body(sl_r, sh_r, c_r, x0_r, sx_r, wa_r, wb_r, wc_r, wd_r, b1_r,
             w2_r, b2_r, out_r):
        r = 1.0 / jnp.maximum(c_r[...], 1.0)
        o2l = sl_r[...] * r
        o2h = sh_r[...] * r
        h = jnp.maximum(
            jnp.dot(o2l, wa_r[...], preferred_element_type=jnp.float32)
            + jnp.dot(o2h, wb_r[...], preferred_element_type=jnp.float32)
            + jnp.dot(x0_r[...], wc_r[...], preferred_element_type=jnp.float32)
            + jnp.dot(sx_r[...], wd_r[...], preferred_element_type=jnp.float32)
            + b1_r[...], 0.0)
        out_r[...] = (
            jnp.dot(h, w2_r[...], preferred_element_type=jnp.float32)
            + b2_r[...])

    full = lambda s: pl.BlockSpec(s, lambda i: (0, 0))
    return pl.pallas_call(
        body,
        grid=(grid,),
        in_specs=[
            pl.BlockSpec((bn, 32), lambda i: (i, 0)),
            pl.BlockSpec((bn, 32), lambda i: (i, 0)),
            pl.BlockSpec((bn, 1), lambda i: (i, 0)),
            pl.BlockSpec((bn, 32), lambda i: (i, 0)),
            pl.BlockSpec((bn, 8), lambda i: (i, 0)),
            full((32, 16)), full((32, 16)), full((32, 16)), full((8, 16)),
            full((1, 16)), full((16, 1)), full((1, 1)),
        ],
        out_specs=pl.BlockSpec((bn, 1), lambda i: (i, 0)),
        out_shape=jax.ShapeDtypeStruct((n, 1), jnp.float32),
    )(s2[0], s2[1], cnt.reshape(n, 1), x0, sx,
      mW1[:32], mW1[32:64], mW1[64:96], mW1[96:104], mb1.reshape(1, 16),
      mW2, mb2.reshape(1, 1))


def _sc_gather(table, src):
    e = src.shape[0]
    d = table.shape[1]
    ew = e // NW
    c = 80
    steps = ew // c
    mesh = plsc.VectorSubcoreMesh(core_axis_name="c", subcore_axis_name="s")

    @functools.partial(
        pl.kernel,
        out_type=jax.ShapeDtypeStruct((e, d), jnp.float32),
        mesh=mesh,
        compiler_params=pltpu.CompilerParams(use_tc_tiling_on_sc=False),
        scratch_types=[
            pltpu.VMEM((c,), jnp.int32),
            pltpu.VMEM((c, d), jnp.float32),
            pltpu.SemaphoreType.DMA,
        ],
    )
    def k(table_hbm, src_hbm, out_hbm, idx_v, rows_v, sem):
        wid = lax.axis_index("s") * NC + lax.axis_index("c")
        base = wid * ew

        def step(i, carry):
            off = base + i * c
            pltpu.sync_copy(src_hbm.at[pl.ds(off, c)], idx_v)
            pltpu.async_copy(table_hbm.at[idx_v], rows_v, sem).wait()
            pltpu.sync_copy(rows_v, out_hbm.at[pl.ds(off, c)])
            return carry

        lax.fori_loop(0, steps, step, 0)

    return k(table, src)


def _sc_segsum(y2, dst, n, with_count):
    e = dst.shape[0]
    ew = e // NS
    c = 80
    steps = ew // c
    rows_t = n // NS
    mesh = plsc.VectorSubcoreMesh(core_axis_name="c", subcore_axis_name="s")

    zeros2 = jnp.zeros((n, 32), jnp.float32)
    zeros8 = jnp.zeros((n, 8), jnp.float32)
    ones_c = jnp.ones((c, 8), jnp.float32)

    out_type = [jax.ShapeDtypeStruct((2, n, 32), jnp.float32)]
    scratch = [
        pltpu.VMEM((c,), jnp.int32),
        pltpu.VMEM((c, 32), jnp.float32),
        pltpu.VMEM_SHARED((n, 32), jnp.float32),
    ]
    if with_count:
        out_type.append(jax.ShapeDtypeStruct((n, 8), jnp.float32))
        scratch += [pltpu.VMEM((c, 8), jnp.float32),
                    pltpu.VMEM_SHARED((n, 8), jnp.float32)]

    @functools.partial(
        pl.kernel,
        out_type=tuple(out_type),
        mesh=mesh,
        compiler_params=pltpu.CompilerParams(use_tc_tiling_on_sc=False),
        scratch_types=scratch,
    )
    def k(y_hbm, dst_hbm, z2_hbm, z8_hbm, one_hbm, *rest):
        if with_count:
            s_hbm, cnt_hbm, idx_v, val_v, acc, one_v, cacc = rest
        else:
            s_hbm, idx_v, val_v, acc = rest
        ci = lax.axis_index("c")
        si = lax.axis_index("s")

        pltpu.sync_copy(z2_hbm.at[pl.ds(si * rows_t, rows_t)],
                        acc.at[pl.ds(si * rows_t, rows_t)])
        if with_count:
            pltpu.sync_copy(z8_hbm.at[pl.ds(si * rows_t, rows_t)],
                            cacc.at[pl.ds(si * rows_t, rows_t)])
            pltpu.sync_copy(one_hbm, one_v)
        plsc.subcore_barrier()

        base = si * ew

        def step(i, carry):
            off = base + i * c
            pltpu.sync_copy(dst_hbm.at[pl.ds(off, c)], idx_v)
            pltpu.sync_copy(y_hbm.at[ci, pl.ds(off, c)], val_v)
            pltpu.sync_copy(val_v, acc.at[idx_v], add=True)
            if with_count:
                pltpu.sync_copy(one_v, cacc.at[idx_v], add=True)
            return carry

        lax.fori_loop(0, steps, step, 0)
        plsc.subcore_barrier()

        pltpu.sync_copy(acc.at[pl.ds(si * rows_t, rows_t)],
                        s_hbm.at[ci, pl.ds(si * rows_t, rows_t)])
        if with_count:
            pltpu.sync_copy(cacc.at[pl.ds(si * rows_t, rows_t)],
                            cnt_hbm.at[pl.ds(si * rows_t, rows_t)])

    res = k(y2, dst, zeros2, zeros8, ones_c)
    if with_count:
        return res[0], res[1][:, 0]
    arr = res[0] if isinstance(res, (tuple, list)) else res
    return arr, None


def kernel(x, sx, edge_index, edge_attr, tW1, tb1, tW2, tb2, sA1, sa1, sA2,
           sa2, sB1, sb1, sB2, sb2, mW1, mb1, mW2, mb2):
    n = x.shape[0]
    src, dst = edge_index[0], edge_index[1]

    x0, p1 = _dense_pre(x, sx, tW1, tb1, tW2, tb2, sA1, sa1)

    g1 = _sc_gather(p1, src)
    y1 = _edge_mlp(g1, edge_attr, sA1[40:44], jnp.zeros((32,), jnp.float32),
                   sA2, sa2)
    s1, cnt = _sc_segsum(y1, dst, n, with_count=True)

    p2 = _dense_mid(s1, cnt, x0, sx, sB1, sb1)

    g2 = _sc_gather(p2, src)
    y2 = _edge_mlp(g2, edge_attr, sB1[104:108], jnp.zeros((64,), jnp.float32),
                   sB2, sb2)
    s2, _ = _sc_segsum(y2, dst, n, with_count=False)

    return _dense_head(s2, cnt, x0, sx, mW1, mb1, mW2, mb2)


# speedup vs baseline: 2.5376x; 1.4889x over previous
import functools

import jax
import jax.numpy as jnp
from jax import lax
from jax.experimental import pallas as pl
from jax.experimental.pallas import tpu as pltpu
from jax.experimental.pallas import tpu_sc as plsc

NC, NS = 2, 16
NW = NC * NS


def _dense_pre(x, sx, tW1, tb1, tW2, tb2, sA1, sa1):
    n = x.shape[0]
    bn = 2000
    grid = n // bn

    def body(x_r, sx_r, w1a_r, w1b_r, b1_r, w2_r, b2_r, a1a_r, a1b_r, ba_r,
             x0_r, p1_r):
        xb = x_r[...]
        sxb = sx_r[...]
        h = jnp.maximum(
            jnp.dot(xb, w1a_r[...], preferred_element_type=jnp.float32)
            + jnp.dot(sxb, w1b_r[...], preferred_element_type=jnp.float32)
            + b1_r[...], 0.0)
        x0 = jnp.maximum(
            jnp.dot(h, w2_r[...], preferred_element_type=jnp.float32)
            + b2_r[...], 0.0)
        p1 = (jnp.dot(x0, a1a_r[...], preferred_element_type=jnp.float32)
              + jnp.dot(sxb, a1b_r[...], preferred_element_type=jnp.float32)
              + ba_r[...])
        x0_r[...] = x0
        p1_r[...] = p1

    full = lambda s: pl.BlockSpec(s, lambda i: (0, 0))
    return pl.pallas_call(
        body,
        grid=(grid,),
        in_specs=[
            pl.BlockSpec((bn, 28), lambda i: (i, 0)),
            pl.BlockSpec((bn, 8), lambda i: (i, 0)),
            full((28, 32)), full((8, 32)), full((1, 32)),
            full((32, 32)), full((1, 32)),
            full((32, 32)), full((8, 32)), full((1, 32)),
        ],
        out_specs=[
            pl.BlockSpec((bn, 32), lambda i: (i, 0)),
            pl.BlockSpec((bn, 32), lambda i: (i, 0)),
        ],
        out_shape=[
            jax.ShapeDtypeStruct((n, 32), jnp.float32),
            jax.ShapeDtypeStruct((n, 32), jnp.float32),
        ],
    )(x, sx, tW1[:28], tW1[28:], tb1.reshape(1, 32), tW2, tb2.reshape(1, 32),
      sA1[:32], sA1[32:40], sa1.reshape(1, 32))


def _edge_mlp(g, edge_attr, w1e, b1, w2, b2):
    e, d = g.shape
    be = 8000
    grid = e // be

    def body(g_r, ea_r, w1e_r, b1_r, w2_r, b2_r, y_r):
        z = jnp.maximum(
            g_r[...]
            + jnp.dot(ea_r[...], w1e_r[...], preferred_element_type=jnp.float32)
            + b1_r[...], 0.0)
        y = jnp.maximum(
            jnp.dot(z, w2_r[...], preferred_element_type=jnp.float32)
            + b2_r[...], 0.0)
        y_r[0] = y[:, :32]
        y_r[1] = y[:, 32:]

    full = lambda s: pl.BlockSpec(s, lambda i: (0, 0))
    return pl.pallas_call(
        body,
        grid=(grid,),
        in_specs=[
            pl.BlockSpec((be, d), lambda i: (i, 0)),
            pl.BlockSpec((be, 4), lambda i: (i, 0)),
            full((4, d)), full((1, d)), full((d, 64)), full((1, 64)),
        ],
        out_specs=pl.BlockSpec((2, be, 32), lambda i: (0, i, 0)),
        out_shape=jax.ShapeDtypeStruct((2, e, 32), jnp.float32),
    )(g, edge_attr, w1e, b1.reshape(1, d), w2, b2.reshape(1, 64))


def _dense_mid(s1, cnt, x0, sx, sB1, sb1):
    n = x0.shape[0]
    bn = 2000
    grid = n // bn

    def body(sl_r, sh_r, c_r, x0_r, sx_r, wa_r, wb_r, wc_r, wd_r, b_r, p2_r):
        r = 1.0 / jnp.maximum(c_r[...], 1.0)
        o1l = sl_r[...] * r
        o1h = sh_r[...] * r
        p2_r[...] = (
            jnp.dot(o1l, wa_r[...], preferred_element_type=jnp.float32)
            + jnp.dot(o1h, wb_r[...], preferred_element_type=jnp.float32)
            + jnp.dot(x0_r[...], wc_r[...], preferred_element_type=jnp.float32)
            + jnp.dot(sx_r[...], wd_r[...], preferred_element_type=jnp.float32)
            + b_r[...])

    full = lambda s: pl.BlockSpec(s, lambda i: (0, 0))
    return pl.pallas_call(
        body,
        grid=(grid,),
        in_specs=[
            pl.BlockSpec((bn, 32), lambda i: (i, 0)),
            pl.BlockSpec((bn, 32), lambda i: (i, 0)),
            pl.BlockSpec((bn, 1), lambda i: (i, 0)),
            pl.BlockSpec((bn, 32), lambda i: (i, 0)),
            pl.BlockSpec((bn, 8), lambda i: (i, 0)),
            full((32, 64)), full((32, 64)), full((32, 64)), full((8, 64)),
            full((1, 64)),
        ],
        out_specs=pl.BlockSpec((bn, 64), lambda i: (i, 0)),
        out_shape=jax.ShapeDtypeStruct((n, 64), jnp.float32),
    )(s1[0], s1[1], cnt.reshape(n, 1), x0, sx,
      sB1[:32], sB1[32:64], sB1[64:96], sB1[96:104], sb1.reshape(1, 64))


def _dense_head(s2, cnt, x0, sx, mW1, mb1, mW2, mb2):
    n = x0.shape[0]
    bn = 2000
    grid = n // bn

    def body(sl_r, sh_r, c_r, x0_r, sx_r, wa_r, wb_r, wc_r, wd_r, b1_r,
             w2_r, b2_r, out_r):
        r = 1.0 / jnp.maximum(c_r[...], 1.0)
        o2l = sl_r[...] * r
        o2h = sh_r[...] * r
        h = jnp.maximum(
            jnp.dot(o2l, wa_r[...], preferred_element_type=jnp.float32)
            + jnp.dot(o2h, wb_r[...], preferred_element_type=jnp.float32)
            + jnp.dot(x0_r[...], wc_r[...], preferred_element_type=jnp.float32)
            + jnp.dot(sx_r[...], wd_r[...], preferred_element_type=jnp.float32)
            + b1_r[...], 0.0)
        out_r[...] = (
            jnp.dot(h, w2_r[...], preferred_element_type=jnp.float32)
            + b2_r[...])

    full = lambda s: pl.BlockSpec(s, lambda i: (0, 0))
    return pl.pallas_call(
        body,
        grid=(grid,),
        in_specs=[
            pl.BlockSpec((bn, 32), lambda i: (i, 0)),
            pl.BlockSpec((bn, 32), lambda i: (i, 0)),
            pl.BlockSpec((bn, 1), lambda i: (i, 0)),
            pl.BlockSpec((bn, 32), lambda i: (i, 0)),
            pl.BlockSpec((bn, 8), lambda i: (i, 0)),
            full((32, 16)), full((32, 16)), full((32, 16)), full((8, 16)),
            full((1, 16)), full((16, 1)), full((1, 1)),
        ],
        out_specs=pl.BlockSpec((bn, 1), lambda i: (i, 0)),
        out_shape=jax.ShapeDtypeStruct((n, 1), jnp.float32),
    )(s2[0], s2[1], cnt.reshape(n, 1), x0, sx,
      mW1[:32], mW1[32:64], mW1[64:96], mW1[96:104], mb1.reshape(1, 16),
      mW2, mb2.reshape(1, 1))


def _sc_gather(table, src):
    e = src.shape[0]
    d = table.shape[1]
    ew = e // NW
    c = 1000
    steps = ew // c
    mesh = plsc.VectorSubcoreMesh(core_axis_name="c", subcore_axis_name="s")

    @functools.partial(
        pl.kernel,
        out_type=jax.ShapeDtypeStruct((e, d), jnp.float32),
        mesh=mesh,
        compiler_params=pltpu.CompilerParams(use_tc_tiling_on_sc=False),
        scratch_types=[
            pltpu.VMEM((c,), jnp.int32),
            pltpu.VMEM((c, d), jnp.float32),
            pltpu.SemaphoreType.DMA,
        ],
    )
    def k(table_hbm, src_hbm, out_hbm, idx_v, rows_v, sem):
        wid = lax.axis_index("s") * NC + lax.axis_index("c")
        base = wid * ew

        def step(i, carry):
            off = base + i * c
            pltpu.sync_copy(src_hbm.at[pl.ds(off, c)], idx_v)
            pltpu.async_copy(table_hbm.at[idx_v], rows_v, sem).wait()
            pltpu.sync_copy(rows_v, out_hbm.at[pl.ds(off, c)])
            return carry

        lax.fori_loop(0, steps, step, 0)

    return k(table, src)


def _sc_segsum(y2, dst, n):
    e = dst.shape[0]
    ew = e // NS
    c = 800
    steps = ew // c
    rows_t = n // NS
    mesh = plsc.VectorSubcoreMesh(core_axis_name="c", subcore_axis_name="s")

    zeros2 = jnp.zeros((n, 32), jnp.float32)

    @functools.partial(
        pl.kernel,
        out_type=jax.ShapeDtypeStruct((2, n, 32), jnp.float32),
        mesh=mesh,
        compiler_params=pltpu.CompilerParams(use_tc_tiling_on_sc=False),
        scratch_types=[
            pltpu.VMEM((c,), jnp.int32),
            pltpu.VMEM((c, 32), jnp.float32),
            pltpu.VMEM_SHARED((n, 32), jnp.float32),
        ],
    )
    def k(y_hbm, dst_hbm, z2_hbm, s_hbm, idx_v, val_v, acc):
        ci = lax.axis_index("c")
        si = lax.axis_index("s")

        pltpu.sync_copy(z2_hbm.at[pl.ds(si * rows_t, rows_t)],
                        acc.at[pl.ds(si * rows_t, rows_t)])
        plsc.subcore_barrier()

        base = si * ew

        def step(i, carry):
            off = base + i * c
            pltpu.sync_copy(dst_hbm.at[pl.ds(off, c)], idx_v)
            pltpu.sync_copy(y_hbm.at[ci, pl.ds(off, c)], val_v)
            pltpu.sync_copy(val_v, acc.at[idx_v], add=True)
            return carry

        lax.fori_loop(0, steps, step, 0)
        plsc.subcore_barrier()

        pltpu.sync_copy(acc.at[pl.ds(si * rows_t, rows_t)],
                        s_hbm.at[ci, pl.ds(si * rows_t, rows_t)])

    return k(y2, dst, zeros2)


def _sc_count(dst, n):
    e = dst.shape[0]
    ew = e // NS
    c = 2000
    steps = ew // c
    rows_t = n // NS
    mesh = plsc.VectorSubcoreMesh(core_axis_name="c", subcore_axis_name="s")

    zeros8 = jnp.zeros((n, 8), jnp.float32)
    ones_c = jnp.ones((c, 8), jnp.float32)

    @functools.partial(
        pl.kernel,
        out_type=jax.ShapeDtypeStruct((n, 8), jnp.float32),
        mesh=mesh,
        compiler_params=pltpu.CompilerParams(use_tc_tiling_on_sc=False),
        scratch_types=[
            pltpu.VMEM((c,), jnp.int32),
            pltpu.VMEM((c, 8), jnp.float32),
            pltpu.VMEM_SHARED((n, 8), jnp.float32),
        ],
    )
    def k(dst_hbm, z8_hbm, one_hbm, cnt_hbm, idx_v, one_v, cacc):
        si = lax.axis_index("s")

        pltpu.sync_copy(z8_hbm.at[pl.ds(si * rows_t, rows_t)],
                        cacc.at[pl.ds(si * rows_t, rows_t)])
        pltpu.sync_copy(one_hbm, one_v)
        plsc.subcore_barrier()

        base = si * ew

        def step(i, carry):
            off = base + i * c
            pltpu.sync_copy(dst_hbm.at[pl.ds(off, c)], idx_v)
            pltpu.sync_copy(one_v, cacc.at[idx_v], add=True)
            return carry

        lax.fori_loop(0, steps, step, 0)
        plsc.subcore_barrier()

        pltpu.sync_copy(cacc.at[pl.ds(si * rows_t, rows_t)],
                        cnt_hbm.at[pl.ds(si * rows_t, rows_t)])

    return k(dst, zeros8, ones_c)[:, 0]


def kernel(x, sx, edge_index, edge_attr, tW1, tb1, tW2, tb2, sA1, sa1, sA2,
           sa2, sB1, sb1, sB2, sb2, mW1, mb1, mW2, mb2):
    n = x.shape[0]
    src, dst = edge_index[0], edge_index[1]

    x0, p1 = _dense_pre(x, sx, tW1, tb1, tW2, tb2, sA1, sa1)

    g1 = _sc_gather(p1, src)
    y1 = _edge_mlp(g1, edge_attr, sA1[40:44], jnp.zeros((32,), jnp.float32),
                   sA2, sa2)
    s1 = _sc_segsum(y1, dst, n)
    cnt = _sc_count(dst, n)

    p2 = _dense_mid(s1, cnt, x0, sx, sB1, sb1)

    g2 = _sc_gather(p2, src)
    y2 = _edge_mlp(g2, edge_attr, sB1[104:108], jnp.zeros((64,), jnp.float32),
                   sB2, sb2)
    s2 = _sc_segsum(y2, dst, n)

    return _dense_head(s2, cnt, x0, sx, mW1, mb1, mW2, mb2)


# speedup vs baseline: 3.9313x; 1.5492x over previous
import functools

import jax
import jax.numpy as jnp
from jax import lax
from jax.experimental import pallas as pl
from jax.experimental.pallas import tpu as pltpu
from jax.experimental.pallas import tpu_sc as plsc

NC, NS = 2, 16
NW = NC * NS


def _dense_pre(x, sx, tW1, tb1, tW2, tb2, sA1, sa1):
    n = x.shape[0]
    bn = 2000
    grid = n // bn

    def body(x_r, sx_r, w1a_r, w1b_r, b1_r, w2_r, b2_r, a1a_r, a1b_r, ba_r,
             x0_r, p1_r):
        xb = x_r[...]
        sxb = sx_r[...]
        h = jnp.maximum(
            jnp.dot(xb, w1a_r[...], preferred_element_type=jnp.float32)
            + jnp.dot(sxb, w1b_r[...], preferred_element_type=jnp.float32)
            + b1_r[...], 0.0)
        x0 = jnp.maximum(
            jnp.dot(h, w2_r[...], preferred_element_type=jnp.float32)
            + b2_r[...], 0.0)
        p1 = (jnp.dot(x0, a1a_r[...], preferred_element_type=jnp.float32)
              + jnp.dot(sxb, a1b_r[...], preferred_element_type=jnp.float32)
              + ba_r[...])
        x0_r[...] = x0
        p1_r[...] = p1

    full = lambda s: pl.BlockSpec(s, lambda i: (0, 0))
    return pl.pallas_call(
        body,
        grid=(grid,),
        in_specs=[
            pl.BlockSpec((bn, 28), lambda i: (i, 0)),
            pl.BlockSpec((bn, 8), lambda i: (i, 0)),
            full((28, 32)), full((8, 32)), full((1, 32)),
            full((32, 32)), full((1, 32)),
            full((32, 32)), full((8, 32)), full((1, 32)),
        ],
        out_specs=[
            pl.BlockSpec((bn, 32), lambda i: (i, 0)),
            pl.BlockSpec((bn, 32), lambda i: (i, 0)),
        ],
        out_shape=[
            jax.ShapeDtypeStruct((n, 32), jnp.float32),
            jax.ShapeDtypeStruct((n, 32), jnp.float32),
        ],
    )(x, sx, tW1[:28], tW1[28:], tb1.reshape(1, 32), tW2, tb2.reshape(1, 32),
      sA1[:32], sA1[32:40], sa1.reshape(1, 32))


def _edge_mlp(g, edge_attr, w1e, w2, b2):
    e, d = g.shape
    e4 = e // 4
    beq = 2000
    grid = e4 // beq
    eye4 = jnp.eye(4, dtype=jnp.float32)
    w1bd = jnp.kron(eye4, w1e)
    w2lo = jnp.kron(eye4, w2[:, :32])
    w2hi = jnp.kron(eye4, w2[:, 32:])
    blo = jnp.tile(b2[:32], 4).reshape(1, 128)
    bhi = jnp.tile(b2[32:], 4).reshape(1, 128)
    gp = g.reshape(e4, 4 * d)
    ea16 = edge_attr.reshape(e4, 16)

    def body(g_r, ea_r, w1_r, w2lo_r, w2hi_r, blo_r, bhi_r, y_r):
        z = jnp.maximum(
            g_r[...]
            + jnp.dot(ea_r[...], w1_r[...], preferred_element_type=jnp.float32),
            0.0)
        y_r[0] = jnp.maximum(
            jnp.dot(z, w2lo_r[...], preferred_element_type=jnp.float32)
            + blo_r[...], 0.0)
        y_r[1] = jnp.maximum(
            jnp.dot(z, w2hi_r[...], preferred_element_type=jnp.float32)
            + bhi_r[...], 0.0)

    full = lambda s: pl.BlockSpec(s, lambda i: (0, 0))
    return pl.pallas_call(
        body,
        grid=(grid,),
        in_specs=[
            pl.BlockSpec((beq, 4 * d), lambda i: (i, 0)),
            pl.BlockSpec((beq, 16), lambda i: (i, 0)),
            full((16, 4 * d)), full((4 * d, 128)), full((4 * d, 128)),
            full((1, 128)), full((1, 128)),
        ],
        out_specs=pl.BlockSpec((2, beq, 128), lambda i: (0, i, 0)),
        out_shape=jax.ShapeDtypeStruct((2, e4, 128), jnp.float32),
    )(gp, ea16, w1bd, w2lo, w2hi, blo, bhi)


def _dense_mid(s1, cnt, x0, sx, sB1, sb1):
    n = x0.shape[0]
    bn = 2000
    grid = n // bn

    def body(sl_r, sh_r, c_r, x0_r, sx_r, wa_r, wb_r, wc_r, wd_r, b_r, p2_r):
        r = 1.0 / jnp.maximum(c_r[...], 1.0)
        o1l = sl_r[...] * r
        o1h = sh_r[...] * r
        p2_r[...] = (
            jnp.dot(o1l, wa_r[...], preferred_element_type=jnp.float32)
            + jnp.dot(o1h, wb_r[...], preferred_element_type=jnp.float32)
            + jnp.dot(x0_r[...], wc_r[...], preferred_element_type=jnp.float32)
            + jnp.dot(sx_r[...], wd_r[...], preferred_element_type=jnp.float32)
            + b_r[...])

    full = lambda s: pl.BlockSpec(s, lambda i: (0, 0))
    return pl.pallas_call(
        body,
        grid=(grid,),
        in_specs=[
            pl.BlockSpec((bn, 32), lambda i: (i, 0)),
            pl.BlockSpec((bn, 32), lambda i: (i, 0)),
            pl.BlockSpec((bn, 1), lambda i: (i, 0)),
            pl.BlockSpec((bn, 32), lambda i: (i, 0)),
            pl.BlockSpec((bn, 8), lambda i: (i, 0)),
            full((32, 64)), full((32, 64)), full((32, 64)), full((8, 64)),
            full((1, 64)),
        ],
        out_specs=pl.BlockSpec((bn, 64), lambda i: (i, 0)),
        out_shape=jax.ShapeDtypeStruct((n, 64), jnp.float32),
    )(s1[0], s1[1], cnt.reshape(n, 1), x0, sx,
      sB1[:32], sB1[32:64], sB1[64:96], sB1[96:104], sb1.reshape(1, 64))


def _dense_head(s2, cnt, x0, sx, mW1, mb1, mW2, mb2):
    n = x0.shape[0]
    bn = 2000
    grid = n // bn

    def body(sl_r, sh_r, c_r, x0_r, sx_r, wa_r, wb_r, wc_r, wd_r, b1_r,
             w2_r, b2_r, out_r):
        r = 1.0 / jnp.maximum(c_r[...], 1.0)
        o2l = sl_r[...] * r
        o2h = sh_r[...] * r
        h = jnp.maximum(
            jnp.dot(o2l, wa_r[...], preferred_element_type=jnp.float32)
            + jnp.dot(o2h, wb_r[...], preferred_element_type=jnp.float32)
            + jnp.dot(x0_r[...], wc_r[...], preferred_element_type=jnp.float32)
            + jnp.dot(sx_r[...], wd_r[...], preferred_element_type=jnp.float32)
            + b1_r[...], 0.0)
        out_r[...] = (
            jnp.dot(h, w2_r[...], preferred_element_type=jnp.float32)
            + b2_r[...])

    full = lambda s: pl.BlockSpec(s, lambda i: (0, 0))
    return pl.pallas_call(
        body,
        grid=(grid,),
        in_specs=[
            pl.BlockSpec((bn, 32), lambda i: (i, 0)),
            pl.BlockSpec((bn, 32), lambda i: (i, 0)),
            pl.BlockSpec((bn, 1), lambda i: (i, 0)),
            pl.BlockSpec((bn, 32), lambda i: (i, 0)),
            pl.BlockSpec((bn, 8), lambda i: (i, 0)),
            full((32, 16)), full((32, 16)), full((32, 16)), full((8, 16)),
            full((1, 16)), full((16, 1)), full((1, 1)),
        ],
        out_specs=pl.BlockSpec((bn, 1), lambda i: (i, 0)),
        out_shape=jax.ShapeDtypeStruct((n, 1), jnp.float32),
    )(s2[0], s2[1], cnt.reshape(n, 1), x0, sx,
      mW1[:32], mW1[32:64], mW1[64:96], mW1[96:104], mb1.reshape(1, 16),
      mW2, mb2.reshape(1, 1))


def _sc_gather(table, src):
    e = src.shape[0]
    d = table.shape[1]
    ew = e // NW
    c = 1000
    steps = ew // c
    mesh = plsc.VectorSubcoreMesh(core_axis_name="c", subcore_axis_name="s")

    @functools.partial(
        pl.kernel,
        out_type=jax.ShapeDtypeStruct((e, d), jnp.float32),
        mesh=mesh,
        compiler_params=pltpu.CompilerParams(use_tc_tiling_on_sc=False),
        scratch_types=[
            pltpu.VMEM((c,), jnp.int32),
            pltpu.VMEM((c, d), jnp.float32),
            pltpu.SemaphoreType.DMA,
        ],
    )
    def k(table_hbm, src_hbm, out_hbm, idx_v, rows_v, sem):
        wid = lax.axis_index("s") * NC + lax.axis_index("c")
        base = wid * ew

        def step(i, carry):
            off = base + i * c
            pltpu.sync_copy(src_hbm.at[pl.ds(off, c)], idx_v)
            pltpu.async_copy(table_hbm.at[idx_v], rows_v, sem).wait()
            pltpu.sync_copy(rows_v, out_hbm.at[pl.ds(off, c)])
            return carry

        lax.fori_loop(0, steps, step, 0)

    return k(table, src)


def _sc_segsum(y2, dst, n):
    e = dst.shape[0]
    ew = e // NS
    c = 800
    steps = ew // c
    rows_t = n // NS
    mesh = plsc.VectorSubcoreMesh(core_axis_name="c", subcore_axis_name="s")

    zeros2 = jnp.zeros((n, 32), jnp.float32)

    @functools.partial(
        pl.kernel,
        out_type=jax.ShapeDtypeStruct((2, n, 32), jnp.float32),
        mesh=mesh,
        compiler_params=pltpu.CompilerParams(use_tc_tiling_on_sc=False),
        scratch_types=[
            pltpu.VMEM((c,), jnp.int32),
            pltpu.VMEM((c, 32), jnp.float32),
            pltpu.VMEM_SHARED((n, 32), jnp.float32),
        ],
    )
    def k(y_hbm, dst_hbm, z2_hbm, s_hbm, idx_v, val_v, acc):
        ci = lax.axis_index("c")
        si = lax.axis_index("s")

        pltpu.sync_copy(z2_hbm.at[pl.ds(si * rows_t, rows_t)],
                        acc.at[pl.ds(si * rows_t, rows_t)])
        plsc.subcore_barrier()

        base = si * ew

        def step(i, carry):
            off = base + i * c
            pltpu.sync_copy(dst_hbm.at[pl.ds(off, c)], idx_v)
            pltpu.sync_copy(y_hbm.at[ci, pl.ds(off, c)], val_v)
            pltpu.sync_copy(val_v, acc.at[idx_v], add=True)
            return carry

        lax.fori_loop(0, steps, step, 0)
        plsc.subcore_barrier()

        pltpu.sync_copy(acc.at[pl.ds(si * rows_t, rows_t)],
                        s_hbm.at[ci, pl.ds(si * rows_t, rows_t)])

    return k(y2, dst, zeros2)


def _sc_count(dst, n):
    e = dst.shape[0]
    ew = e // NS
    c = 2000
    steps = ew // c
    rows_t = n // NS
    mesh = plsc.VectorSubcoreMesh(core_axis_name="c", subcore_axis_name="s")

    zeros8 = jnp.zeros((n, 8), jnp.float32)
    ones_c = jnp.ones((c, 8), jnp.float32)

    @functools.partial(
        pl.kernel,
        out_type=jax.ShapeDtypeStruct((n, 8), jnp.float32),
        mesh=mesh,
        compiler_params=pltpu.CompilerParams(use_tc_tiling_on_sc=False),
        scratch_types=[
            pltpu.VMEM((c,), jnp.int32),
            pltpu.VMEM((c, 8), jnp.float32),
            pltpu.VMEM_SHARED((n, 8), jnp.float32),
        ],
    )
    def k(dst_hbm, z8_hbm, one_hbm, cnt_hbm, idx_v, one_v, cacc):
        si = lax.axis_index("s")

        pltpu.sync_copy(z8_hbm.at[pl.ds(si * rows_t, rows_t)],
                        cacc.at[pl.ds(si * rows_t, rows_t)])
        pltpu.sync_copy(one_hbm, one_v)
        plsc.subcore_barrier()

        base = si * ew

        def step(i, carry):
            off = base + i * c
            pltpu.sync_copy(dst_hbm.at[pl.ds(off, c)], idx_v)
            pltpu.sync_copy(one_v, cacc.at[idx_v], add=True)
            return carry

        lax.fori_loop(0, steps, step, 0)
        plsc.subcore_barrier()

        pltpu.sync_copy(cacc.at[pl.ds(si * rows_t, rows_t)],
                        cnt_hbm.at[pl.ds(si * rows_t, rows_t)])

    return k(dst, zeros8, ones_c)[:, 0]


def kernel(x, sx, edge_index, edge_attr, tW1, tb1, tW2, tb2, sA1, sa1, sA2,
           sa2, sB1, sb1, sB2, sb2, mW1, mb1, mW2, mb2):
    n = x.shape[0]
    e = edge_index.shape[1]
    src, dst = edge_index[0], edge_index[1]

    x0, p1 = _dense_pre(x, sx, tW1, tb1, tW2, tb2, sA1, sa1)

    g1 = _sc_gather(p1, src)
    y1 = _edge_mlp(g1, edge_attr, sA1[40:44], sA2, sa2)
    s1 = _sc_segsum(y1.reshape(2, e, 32), dst, n)
    cnt = _sc_count(dst, n)

    p2 = _dense_mid(s1, cnt, x0, sx, sB1, sb1)

    g2 = _sc_gather(p2, src)
    y2 = _edge_mlp(g2, edge_attr, sB1[104:108], sB2, sb2)
    s2 = _sc_segsum(y2.reshape(2, e, 32), dst, n)

    return _dense_head(s2, cnt, x0, sx, mW1, mb1, mW2, mb2)
